# baseline (speedup 1.0000x reference)
import functools

import jax
import jax.numpy as jnp
from jax import lax
from jax.experimental import pallas as pl
from jax.experimental.pallas import tpu as pltpu
from jax.experimental.pallas import tpu_sc as plsc

_N = 20000
_NS = 16
_PER = 1280
_CH = _PER // 16
_K = 100
_SCORE_THRESH = 0.05
_NMS_THRESH = 0.5
_BIG = 2 ** 30


def _row6(lane, a, b, c, d, e, f):
    z = jnp.zeros((16,), jnp.float32)
    r = jnp.where(lane == 0, a, z)
    r = jnp.where(lane == 1, b, r)
    r = jnp.where(lane == 2, c, r)
    r = jnp.where(lane == 3, d, r)
    r = jnp.where(lane == 4, e, r)
    r = jnp.where(lane == 5, f, r)
    return r


def _nms_sc(x1_h, y1_h, x2_h, y2_h, s_h, out_h,
            x1_v, y1_v, x2_v, y2_v, s_v, stats_v, allstats_v, out_v, shared):
    sid = lax.axis_index("s")
    cid = lax.axis_index("c")
    base = sid * _PER

    pltpu.sync_copy(x1_h.at[pl.ds(base, _PER)], x1_v)
    pltpu.sync_copy(y1_h.at[pl.ds(base, _PER)], y1_v)
    pltpu.sync_copy(x2_h.at[pl.ds(base, _PER)], x2_v)
    pltpu.sync_copy(y2_h.at[pl.ds(base, _PER)], y2_v)
    pltpu.sync_copy(s_h.at[pl.ds(base, _PER)], s_v)

    lane = lax.broadcasted_iota(jnp.int32, (16,), 0)
    neg = jnp.float32(-jnp.inf)
    negv = jnp.full((16,), neg)

    vmax = negv
    vidx = lane
    for c in range(_CH):
        sl = pl.ds(c * 16, 16)
        sr = s_v[sl]
        gidx = lane + (base + c * 16)
        ok = (sr > _SCORE_THRESH) & (gidx < _N)
        sm = jnp.where(ok, sr, negv)
        s_v[sl] = sm
        lidx = lane + c * 16
        if c == 0:
            vmax, vidx = sm, lidx
        else:
            cond = sm > vmax
            vmax = jnp.where(cond, sm, vmax)
            vidx = jnp.where(cond, lidx, vidx)

    def body(i, carry):
        vmax, vidx = carry
        m_l = jnp.max(vmax)
        i_l = jnp.min(jnp.where(vmax == m_l, vidx, jnp.full((16,), _BIG)))
        ginds = jnp.full((16,), i_l, jnp.int32)
        gx1 = plsc.load_gather(x1_v, [ginds])
        gy1 = plsc.load_gather(y1_v, [ginds])
        gx2 = plsc.load_gather(x2_v, [ginds])
        gy2 = plsc.load_gather(y2_v, [ginds])
        gidxf = jnp.full((16,), i_l + base, jnp.int32).astype(jnp.float32)
        stats_v[...] = _row6(lane, jnp.full((16,), m_l), gidxf, gx1, gy1, gx2, gy2)
        pltpu.sync_copy(stats_v, shared.at[pl.ds(sid * 16, 16)])
        plsc.subcore_barrier()
        pltpu.sync_copy(shared, allstats_v)
        plsc.subcore_barrier()

        rows16 = lane * 16
        maxv = plsc.load_gather(allstats_v, [rows16])
        idxv = plsc.load_gather(allstats_v, [rows16 + 1]).astype(jnp.int32)
        wx1 = plsc.load_gather(allstats_v, [rows16 + 2])
        wy1 = plsc.load_gather(allstats_v, [rows16 + 3])
        wx2 = plsc.load_gather(allstats_v, [rows16 + 4])
        wy2 = plsc.load_gather(allstats_v, [rows16 + 5])
        m = jnp.max(maxv)
        has = m > neg
        gi = jnp.min(jnp.where(maxv == m, idxv, jnp.full((16,), _BIG)))
        sel = idxv == jnp.full((16,), gi)
        z = jnp.zeros((16,), jnp.float32)
        bx1 = jnp.sum(jnp.where(sel, wx1, z))
        by1 = jnp.sum(jnp.where(sel, wy1, z))
        bx2 = jnp.sum(jnp.where(sel, wx2, z))
        by2 = jnp.sum(jnp.where(sel, wy2, z))

        orow = _row6(lane,
                     jnp.where(has, bx1, 0.0), jnp.where(has, by1, 0.0),
                     jnp.where(has, bx2, 0.0), jnp.where(has, by2, 0.0),
                     jnp.where(has, m, 0.0), jnp.float32(0.0))
        out_v[pl.ds(i * 16, 16)] = orow

        sx1 = jnp.where(has, bx1, 5000.0)
        sy1 = jnp.where(has, by1, 5000.0)
        sx2 = jnp.where(has, bx2, 4999.0)
        sy2 = jnp.where(has, by2, 4999.0)
        barea = (sx2 - sx1) * (sy2 - sy1)
        bx1v = jnp.full((16,), sx1)
        by1v = jnp.full((16,), sy1)
        bx2v = jnp.full((16,), sx2)
        by2v = jnp.full((16,), sy2)
        bareav = jnp.full((16,), barea)
        lgiv = jnp.full((16,), gi - base, jnp.int32)

        nvmax = negv
        nvidx = lane
        for c in range(_CH):
            sl = pl.ds(c * 16, 16)
            cx1 = x1_v[sl]
            cy1 = y1_v[sl]
            cx2 = x2_v[sl]
            cy2 = y2_v[sl]
            cs = s_v[sl]
            ix1 = jnp.maximum(bx1v, cx1)
            iy1 = jnp.maximum(by1v, cy1)
            ix2 = jnp.minimum(bx2v, cx2)
            iy2 = jnp.minimum(by2v, cy2)
            inter = jnp.maximum(ix2 - ix1, 0.0) * jnp.maximum(iy2 - iy1, 0.0)
            area = (cx2 - cx1) * (cy2 - cy1)
            union = bareav + area - inter
            iou = inter / jnp.maximum(union, 1e-8)
            lidx = lane + c * 16
            supp = (iou > _NMS_THRESH) | (lidx == lgiv)
            snew = jnp.where(supp, negv, cs)
            s_v[sl] = snew
            if c == 0:
                nvmax, nvidx = snew, lidx
            else:
                cond = snew > nvmax
                nvmax = jnp.where(cond, snew, nvmax)
                nvidx = jnp.where(cond, lidx, nvidx)
        return (nvmax, nvidx)

    lax.fori_loop(0, _K, body, (vmax, vidx))

    @pl.when((sid == 0) & (cid == 0))
    def _():
        pltpu.sync_copy(out_v, out_h)


def kernel(boxes, scores):
    pad = _NS * _PER - _N
    bt = jnp.pad(jnp.transpose(boxes), ((0, 0), (0, pad)))
    s = jnp.pad(scores, (0, pad))

    mesh = plsc.VectorSubcoreMesh(
        core_axis_name="c", subcore_axis_name="s", num_cores=2)
    f = functools.partial(
        pl.kernel,
        mesh=mesh,
        compiler_params=pltpu.CompilerParams(needs_layout_passes=False),
        out_type=jax.ShapeDtypeStruct((_K * 16,), jnp.float32),
        scratch_types=[
            pltpu.VMEM((_PER,), jnp.float32),
            pltpu.VMEM((_PER,), jnp.float32),
            pltpu.VMEM((_PER,), jnp.float32),
            pltpu.VMEM((_PER,), jnp.float32),
            pltpu.VMEM((_PER,), jnp.float32),
            pltpu.VMEM((16,), jnp.float32),
            pltpu.VMEM((256,), jnp.float32),
            pltpu.VMEM((_K * 16,), jnp.float32),
            pltpu.VMEM_SHARED((256,), jnp.float32),
        ],
    )(_nms_sc)
    out = f(bt[0], bt[1], bt[2], bt[3], s)
    return out.reshape(_K, 16)[:, :5]


# speedup vs baseline: 1.0928x; 1.0928x over previous
import functools

import jax
import jax.numpy as jnp
from jax import lax
from jax.experimental import pallas as pl
from jax.experimental.pallas import tpu as pltpu
from jax.experimental.pallas import tpu_sc as plsc

_N = 20000
_NS = 16
_PER = 1280
_CH = _PER // 16
_K = 100
_SCORE_THRESH = 0.05
_NMS_THRESH = 0.5
_BIG = 2 ** 30


def _row6(lane, a, b, c, d, e, f):
    z = jnp.zeros((16,), jnp.float32)
    r = jnp.where(lane == 0, a, z)
    r = jnp.where(lane == 1, b, r)
    r = jnp.where(lane == 2, c, r)
    r = jnp.where(lane == 3, d, r)
    r = jnp.where(lane == 4, e, r)
    r = jnp.where(lane == 5, f, r)
    return r


def _nms_sc(x1_h, y1_h, x2_h, y2_h, s_h, out_h,
            x1_v, y1_v, x2_v, y2_v, s_v, area_v, stats_v, allstats_v, out_v,
            shared):
    sid = lax.axis_index("s")
    cid = lax.axis_index("c")
    base = sid * _PER

    pltpu.sync_copy(x1_h.at[pl.ds(base, _PER)], x1_v)
    pltpu.sync_copy(y1_h.at[pl.ds(base, _PER)], y1_v)
    pltpu.sync_copy(x2_h.at[pl.ds(base, _PER)], x2_v)
    pltpu.sync_copy(y2_h.at[pl.ds(base, _PER)], y2_v)
    pltpu.sync_copy(s_h.at[pl.ds(base, _PER)], s_v)

    lane = lax.broadcasted_iota(jnp.int32, (16,), 0)
    neg = jnp.float32(-jnp.inf)
    negv = jnp.full((16,), neg)

    vmax = negv
    vidx = lane
    for c in range(_CH):
        sl = pl.ds(c * 16, 16)
        sr = s_v[sl]
        gidx = lane + (base + c * 16)
        ok = (sr > _SCORE_THRESH) & (gidx < _N)
        sm = jnp.where(ok, sr, negv)
        s_v[sl] = sm
        area_v[sl] = (x2_v[sl] - x1_v[sl]) * (y2_v[sl] - y1_v[sl])
        lidx = lane + c * 16
        if c == 0:
            vmax, vidx = sm, lidx
        else:
            cond = sm > vmax
            vmax = jnp.where(cond, sm, vmax)
            vidx = jnp.where(cond, lidx, vidx)

    def body(i, carry):
        vmax, vidx = carry
        m_l = jnp.max(vmax)
        i_l = jnp.min(jnp.where(vmax == m_l, vidx, jnp.full((16,), _BIG)))
        ginds = jnp.full((16,), i_l, jnp.int32)
        gx1 = plsc.load_gather(x1_v, [ginds])
        gy1 = plsc.load_gather(y1_v, [ginds])
        gx2 = plsc.load_gather(x2_v, [ginds])
        gy2 = plsc.load_gather(y2_v, [ginds])
        gidxf = jnp.full((16,), i_l + base, jnp.int32).astype(jnp.float32)
        stats_v[...] = _row6(lane, jnp.full((16,), m_l), gidxf, gx1, gy1, gx2, gy2)
        off = (i % 2) * 256
        pltpu.sync_copy(stats_v, shared.at[pl.ds(off + sid * 16, 16)])
        plsc.subcore_barrier()
        pltpu.sync_copy(shared.at[pl.ds(off, 256)], allstats_v)

        rows16 = lane * 16
        maxv = plsc.load_gather(allstats_v, [rows16])
        idxv = plsc.load_gather(allstats_v, [rows16 + 1]).astype(jnp.int32)
        m = jnp.max(maxv)
        gi = jnp.min(jnp.where(maxv == m, idxv, jnp.full((16,), _BIG)))
        w16 = (gi // _PER) * 16
        bx1v = plsc.load_gather(allstats_v, [jnp.full((16,), w16 + 2, jnp.int32)])
        by1v = plsc.load_gather(allstats_v, [jnp.full((16,), w16 + 3, jnp.int32)])
        bx2v = plsc.load_gather(allstats_v, [jnp.full((16,), w16 + 4, jnp.int32)])
        by2v = plsc.load_gather(allstats_v, [jnp.full((16,), w16 + 5, jnp.int32)])
        mv = jnp.full((16,), m)
        hasv = mv > negv
        z = jnp.zeros((16,), jnp.float32)

        orow = jnp.where(hasv, _row6(lane, bx1v, by1v, bx2v, by2v, mv, z), z)
        out_v[pl.ds(i * 16, 16)] = orow

        sx1 = jnp.where(hasv, bx1v, jnp.full((16,), 5000.0))
        sy1 = jnp.where(hasv, by1v, jnp.full((16,), 5000.0))
        sx2 = jnp.where(hasv, bx2v, jnp.full((16,), 4999.0))
        sy2 = jnp.where(hasv, by2v, jnp.full((16,), 4999.0))
        bareav = (sx2 - sx1) * (sy2 - sy1)

        nvmax = negv
        nvidx = lane
        for c in range(_CH):
            sl = pl.ds(c * 16, 16)
            cx1 = x1_v[sl]
            cy1 = y1_v[sl]
            cx2 = x2_v[sl]
            cy2 = y2_v[sl]
            cs = s_v[sl]
            ix1 = jnp.maximum(sx1, cx1)
            iy1 = jnp.maximum(sy1, cy1)
            ix2 = jnp.minimum(sx2, cx2)
            iy2 = jnp.minimum(sy2, cy2)
            inter = jnp.maximum(ix2 - ix1, 0.0) * jnp.maximum(iy2 - iy1, 0.0)
            union = bareav + area_v[sl] - inter
            iou = inter / jnp.maximum(union, 1e-8)
            snew = jnp.where(iou > _NMS_THRESH, negv, cs)
            s_v[sl] = snew
            lidx = lane + c * 16
            if c == 0:
                nvmax, nvidx = snew, lidx
            else:
                cond = snew > nvmax
                nvmax = jnp.where(cond, snew, nvmax)
                nvidx = jnp.where(cond, lidx, nvidx)
        return (nvmax, nvidx)

    lax.fori_loop(0, _K, body, (vmax, vidx))

    @pl.when((sid == 0) & (cid == 0))
    def _():
        pltpu.sync_copy(out_v, out_h)


def kernel(boxes, scores):
    pad = _NS * _PER - _N
    bt = jnp.pad(jnp.transpose(boxes), ((0, 0), (0, pad)))
    s = jnp.pad(scores, (0, pad))

    mesh = plsc.VectorSubcoreMesh(
        core_axis_name="c", subcore_axis_name="s", num_cores=2)
    f = functools.partial(
        pl.kernel,
        mesh=mesh,
        compiler_params=pltpu.CompilerParams(needs_layout_passes=False),
        out_type=jax.ShapeDtypeStruct((_K * 16,), jnp.float32),
        scratch_types=[
            pltpu.VMEM((_PER,), jnp.float32),
            pltpu.VMEM((_PER,), jnp.float32),
            pltpu.VMEM((_PER,), jnp.float32),
            pltpu.VMEM((_PER,), jnp.float32),
            pltpu.VMEM((_PER,), jnp.float32),
            pltpu.VMEM((_PER,), jnp.float32),
            pltpu.VMEM((16,), jnp.float32),
            pltpu.VMEM((256,), jnp.float32),
            pltpu.VMEM((_K * 16,), jnp.float32),
            pltpu.VMEM_SHARED((512,), jnp.float32),
        ],
    )(_nms_sc)
    out = f(bt[0], bt[1], bt[2], bt[3], s)
    return out.reshape(_K, 16)[:, :5]
